# baseline (speedup 1.0000x reference)
import functools

import jax
import jax.numpy as jnp
from jax import lax
from jax.experimental import pallas as pl
from jax.experimental.pallas import tpu as pltpu
from jax.experimental.pallas import tpu_sc as plsc

N = 10000
E = 320000
D = 128
H = 128
C = 10

NC = 2
NS = 16
NW = NC * NS
KD = 128
KDP = KD // 2
TCHD = E // KD
K = 80
CPW = E // K // NW
ACC = 10240
ZR = ACC // NS
ZCH = 128
NSL = 2
NSLS = 2

_mesh = plsc.VectorSubcoreMesh(core_axis_name="c", subcore_axis_name="s")


@functools.partial(
    pl.kernel,
    out_type=[
        jax.ShapeDtypeStruct((NC, ACC), jnp.float32),
        jax.ShapeDtypeStruct((NC, ACC), jnp.float32),
    ],
    mesh=_mesh,
    scratch_types=[
        pltpu.VMEM((KD,), jnp.int32),
        pltpu.VMEM((KD,), jnp.int32),
        pltpu.VMEM((KD,), jnp.int32),
        pltpu.VMEM((KD,), jnp.int32),
        pltpu.VMEM((KDP,), jnp.int32),
        pltpu.VMEM((KDP,), jnp.int32),
        pltpu.VMEM((KDP,), jnp.int32),
        pltpu.VMEM((KDP,), jnp.int32),
        pltpu.VMEM((KD,), jnp.float32),
        pltpu.VMEM((ZR,), jnp.float32),
        pltpu.VMEM_SHARED((ACC,), jnp.float32),
        pltpu.VMEM_SHARED((ACC,), jnp.float32),
        pltpu.SemaphoreType.DMA,
        pltpu.SemaphoreType.DMA,
    ],
)
def _deg_kernel(src_hbm, dst_hbm, ones_hbm, z1_hbm,
                do_out, di_out, sidx0, didx0, sidx1, didx1,
                sp0, dp0, sp1, dp1, ones_v, z1_v,
                do_acc, di_acc, isem0, isem1):
    cid = lax.axis_index("c")
    sid = lax.axis_index("s")
    wid = cid * NS + sid

    pltpu.sync_copy(z1_hbm, z1_v)
    pltpu.sync_copy(z1_v, do_acc.at[pl.ds(sid * ZR, ZR)])
    pltpu.sync_copy(z1_v, di_acc.at[pl.ds(sid * ZR, ZR)])
    pltpu.sync_copy(ones_hbm, ones_v)
    plsc.subcore_barrier()

    start = (wid * TCHD) // NW
    end = ((wid + 1) * TCHD) // NW
    ncw = end - start
    nfull = ncw // NSL
    sidx = (sidx0, sidx1)
    didx = (didx0, didx1)
    sp = (sp0, sp1)
    dp = (dp0, dp1)
    isem = (isem0, isem1)

    def widen_idx(refp, ref32):
        for cc in range(KDP // 16):
            v = refp[pl.ds(cc * 16, 16)]
            ref32[pl.ds(cc * 32, 16)] = v & 0xFFFF
            ref32[pl.ds(cc * 32 + 16, 16)] = lax.shift_right_logical(v, 16)

    def body(t, carry):
        c0 = start + t * NSL
        descs = []
        for k in range(NSL):
            off = pl.multiple_of((c0 + k) * KDP, 8)
            descs.append(pltpu.async_copy(
                src_hbm.at[pl.ds(off, KDP)], sp[k], isem[k]))
            descs.append(pltpu.async_copy(
                dst_hbm.at[pl.ds(off, KDP)], dp[k], isem[k]))
        for k in range(NSL):
            descs[2 * k].wait()
            descs[2 * k + 1].wait()
            widen_idx(sp[k], sidx[k])
            widen_idx(dp[k], didx[k])
            pltpu.sync_copy(ones_v, do_acc.at[sidx[k]], add=True)
            pltpu.sync_copy(ones_v, di_acc.at[didx[k]], add=True)
        return carry

    lax.fori_loop(0, nfull, body, 0)

    @pl.when(ncw - nfull * NSL > 0)
    def _():
        off = pl.multiple_of((start + nfull * NSL) * KDP, 8)
        pltpu.sync_copy(src_hbm.at[pl.ds(off, KDP)], sp0)
        pltpu.sync_copy(dst_hbm.at[pl.ds(off, KDP)], dp0)
        widen_idx(sp0, sidx0)
        widen_idx(dp0, didx0)
        pltpu.sync_copy(ones_v, do_acc.at[sidx0], add=True)
        pltpu.sync_copy(ones_v, di_acc.at[didx0], add=True)

    plsc.subcore_barrier()

    r0 = sid * ZR
    pltpu.sync_copy(do_acc.at[pl.ds(r0, ZR)], do_out.at[cid, pl.ds(r0, ZR)])
    pltpu.sync_copy(di_acc.at[pl.ds(r0, ZR)], di_out.at[cid, pl.ds(r0, ZR)])


@functools.partial(
    pl.kernel,
    out_type=[
        jax.ShapeDtypeStruct((NC, ACC, D), jnp.float32),
        jax.ShapeDtypeStruct((NC, ACC), jnp.float32),
    ],
    mesh=_mesh,
    scratch_types=[
        pltpu.VMEM((K,), jnp.int32),
        pltpu.VMEM((K,), jnp.int32),
        pltpu.VMEM((K,), jnp.int32),
        pltpu.VMEM((K,), jnp.int32),
        pltpu.VMEM((K, D), jnp.float32),
        pltpu.VMEM((K, D), jnp.float32),
        pltpu.VMEM((K,), jnp.float32),
        pltpu.VMEM((K,), jnp.float32),
        pltpu.VMEM((ZCH, D), jnp.float32),
        pltpu.VMEM((ZR,), jnp.float32),
        pltpu.VMEM_SHARED((ACC, D), jnp.float32),
        pltpu.VMEM_SHARED((ACC,), jnp.float32),
        pltpu.SemaphoreType.DMA,
        pltpu.SemaphoreType.DMA,
        pltpu.SemaphoreType.DMA,
        pltpu.SemaphoreType.DMA,
        pltpu.SemaphoreType.DMA,
        pltpu.SemaphoreType.DMA,
    ],
)
def _spmm_kernel(xp_hbm, ndst_hbm, src_hbm, dst_hbm, zrows_hbm, z1_hbm,
                 agg_out, c_out,
                 sidx0, didx0, sidx1, didx1, rows0, rows1, nv0, nv1,
                 zb_v, z1_v, agg_acc, c_acc,
                 isem0, isem1, gsem0, gsem1, ssem0, ssem1):
    cid = lax.axis_index("c")
    sid = lax.axis_index("s")
    wid = cid * NS + sid

    pltpu.sync_copy(zrows_hbm, zb_v)
    for z in range(ZR // ZCH):
        pltpu.sync_copy(zb_v, agg_acc.at[pl.ds(sid * ZR + z * ZCH, ZCH)])
    pltpu.sync_copy(z1_hbm, z1_v)
    pltpu.sync_copy(z1_v, c_acc.at[pl.ds(sid * ZR, ZR)])
    plsc.subcore_barrier()

    start = wid * CPW
    nfull = CPW // NSLS
    sidx = (sidx0, sidx1)
    didx = (didx0, didx1)
    rows = (rows0, rows1)
    nv = (nv0, nv1)
    isem = (isem0, isem1)
    gsem = (gsem0, gsem1)
    ssem = (ssem0, ssem1)

    def drain_scatter(k):
        pltpu.make_async_copy(rows[k], agg_acc.at[didx[k]], ssem[k]).wait()
        pltpu.make_async_copy(nv[k], c_acc.at[sidx[k]], ssem[k]).wait()

    def body(t, carry):
        @pl.when(t > 0)
        def _():
            for k in range(NSLS):
                drain_scatter(k)
        c0 = start + t * NSLS
        idescs = []
        for k in range(NSLS):
            off = pl.multiple_of((c0 + k) * K, 8)
            idescs.append(pltpu.async_copy(
                src_hbm.at[pl.ds(off, K)], sidx[k], isem[k]))
            idescs.append(pltpu.async_copy(
                dst_hbm.at[pl.ds(off, K)], didx[k], isem[k]))
        gdescs = []
        for k in range(NSLS):
            idescs[2 * k].wait()
            idescs[2 * k + 1].wait()
            gdescs.append(pltpu.async_copy(
                xp_hbm.at[sidx[k]], rows[k], gsem[k]))
            gdescs.append(pltpu.async_copy(
                ndst_hbm.at[didx[k]], nv[k], gsem[k]))
        for k in range(NSLS):
            gdescs[2 * k].wait()
            gdescs[2 * k + 1].wait()
            pltpu.async_copy(rows[k], agg_acc.at[didx[k]], ssem[k], add=True)
            pltpu.async_copy(nv[k], c_acc.at[sidx[k]], ssem[k], add=True)
        return carry

    lax.fori_loop(0, nfull, body, 0)
    for k in range(NSLS):
        drain_scatter(k)

    for j in range(CPW - (CPW // NSLS) * NSLS):
        off = pl.multiple_of((start + (CPW // NSLS) * NSLS + j) * K, 8)
        pltpu.sync_copy(src_hbm.at[pl.ds(off, K)], sidx0)
        pltpu.sync_copy(dst_hbm.at[pl.ds(off, K)], didx0)
        pltpu.async_copy(xp_hbm.at[sidx0], rows0, gsem0).wait()
        pltpu.sync_copy(ndst_hbm.at[didx0], nv0)
        pltpu.sync_copy(rows0, agg_acc.at[didx0], add=True)
        pltpu.sync_copy(nv0, c_acc.at[sidx0], add=True)

    plsc.subcore_barrier()

    for z in range(ZR // ZCH):
        r0 = sid * ZR + z * ZCH
        pltpu.sync_copy(agg_acc.at[pl.ds(r0, ZCH)],
                        agg_out.at[cid, pl.ds(r0, ZCH)])
    r0 = sid * ZR
    pltpu.sync_copy(c_acc.at[pl.ds(r0, ZR)], c_out.at[cid, pl.ds(r0, ZR)])


def _prep_body(do_ref, di_ref, x_ref, ns_ref, nd_ref, xp_ref):
    do = do_ref[0] + do_ref[1]
    di = di_ref[0] + di_ref[1]
    ns = lax.rsqrt(jnp.maximum(do, 1.0))
    ns_ref[...] = ns
    nd_ref[...] = lax.rsqrt(jnp.maximum(di, 1.0))
    xp_ref[...] = x_ref[...] * ns


BLK = 1280


def _finish_body(agg_ref, nd_ref, ns_ref, c_ref,
                 w1_ref, b1_ref, w2_ref, b2_ref, wc_ref, bc_ref,
                 out_ref, acc_ref):
    i = pl.program_id(0)

    @pl.when(i == 0)
    def _():
        acc_ref[...] = jnp.zeros_like(acc_ref)

    agg = (agg_ref[0] + agg_ref[1]) * nd_ref[...]
    h1 = jnp.dot(agg, w1_ref[...], preferred_element_type=jnp.float32)
    h1 = h1 + b1_ref[...]
    g = jnp.maximum(h1, 0.0) * ns_ref[...]
    cvec = c_ref[0] + c_ref[1]
    acc_ref[...] += jnp.sum(g * cvec, axis=0, keepdims=True)

    @pl.when(i == pl.num_programs(0) - 1)
    def _():
        pooled = acc_ref[...] * (1.0 / N)
        hg = jnp.dot(pooled, w2_ref[...], preferred_element_type=jnp.float32)
        hg = hg + b2_ref[...]
        out_ref[...] = (
            jnp.dot(hg, wc_ref[...], preferred_element_type=jnp.float32)
            + bc_ref[...])


def kernel(in_feat, edge_index, W1, b1, W2, b2, Wc, bc):
    src = edge_index[0]
    dst = edge_index[1]
    ones_k = jnp.ones((KD,), jnp.float32)
    z1 = jnp.zeros((ZR,), jnp.float32)
    zrows = jnp.zeros((ZCH, D), jnp.float32)

    sp = lax.bitcast_convert_type(
        src.astype(jnp.int16).reshape(E // 2, 2), jnp.int32)
    dp = lax.bitcast_convert_type(
        dst.astype(jnp.int16).reshape(E // 2, 2), jnp.int32)

    deg_out_p, deg_in_p = _deg_kernel(sp, dp, ones_k, z1)

    ns, nd, xp = pl.pallas_call(
        _prep_body,
        out_shape=[
            jax.ShapeDtypeStruct((ACC, 1), jnp.float32),
            jax.ShapeDtypeStruct((ACC, 1), jnp.float32),
            jax.ShapeDtypeStruct((ACC, D), jnp.float32),
        ],
    )(deg_out_p.reshape(NC, ACC, 1), deg_in_p.reshape(NC, ACC, 1),
      jnp.pad(in_feat, ((0, ACC - N), (0, 0))))

    agg_p, c_p = _spmm_kernel(xp, nd.reshape(ACC), src, dst, zrows, z1)

    wc_pad = jnp.pad(Wc, ((0, 0), (0, D - C)))
    bc_pad = jnp.pad(bc, ((0, D - C)))

    grid = ACC // BLK
    out_pad = pl.pallas_call(
        _finish_body,
        grid=(grid,),
        in_specs=[
            pl.BlockSpec((NC, BLK, D), lambda i: (0, i, 0)),
            pl.BlockSpec((BLK, 1), lambda i: (i, 0)),
            pl.BlockSpec((BLK, 1), lambda i: (i, 0)),
            pl.BlockSpec((NC, BLK, 1), lambda i: (0, i, 0)),
            pl.BlockSpec((D, H), lambda i: (0, 0)),
            pl.BlockSpec((1, H), lambda i: (0, 0)),
            pl.BlockSpec((H, H), lambda i: (0, 0)),
            pl.BlockSpec((1, H), lambda i: (0, 0)),
            pl.BlockSpec((H, D), lambda i: (0, 0)),
            pl.BlockSpec((1, D), lambda i: (0, 0)),
        ],
        out_specs=pl.BlockSpec((1, D), lambda i: (0, 0)),
        out_shape=jax.ShapeDtypeStruct((1, D), jnp.float32),
        scratch_shapes=[pltpu.VMEM((1, D), jnp.float32)],
    )(agg_p, nd, ns, c_p.reshape(NC, ACC, 1),
      W1, b1.reshape(1, H), W2, b2.reshape(1, H), wc_pad, bc_pad.reshape(1, D))

    return out_pad[:, :C]


# speedup vs baseline: 1.8608x; 1.8608x over previous
import functools

import jax
import jax.numpy as jnp
from jax import lax
from jax.experimental import pallas as pl
from jax.experimental.pallas import tpu as pltpu
from jax.experimental.pallas import tpu_sc as plsc

N = 10000
E = 320000
D = 128
H = 128
C = 10

NC = 2
NS = 16
NW = NC * NS
KD = 128
TCHD = E // KD
K = 80
CPW = E // K // NW
ACC = 10240
ZR = ACC // NS
ZCH = 128
NSL = 2
NSLS = 2

_mesh = plsc.VectorSubcoreMesh(core_axis_name="c", subcore_axis_name="s")


@functools.partial(
    pl.kernel,
    out_type=[
        jax.ShapeDtypeStruct((NC, ACC), jnp.float32),
        jax.ShapeDtypeStruct((NC, ACC), jnp.float32),
    ],
    mesh=_mesh,
    scratch_types=[
        pltpu.VMEM((KD,), jnp.int32),
        pltpu.VMEM((KD,), jnp.int32),
        pltpu.VMEM((KD,), jnp.int32),
        pltpu.VMEM((KD,), jnp.int32),
        pltpu.VMEM((KD,), jnp.float32),
        pltpu.VMEM((ZR,), jnp.float32),
        pltpu.VMEM_SHARED((ACC,), jnp.float32),
        pltpu.VMEM_SHARED((ACC,), jnp.float32),
        pltpu.SemaphoreType.DMA,
        pltpu.SemaphoreType.DMA,
    ],
)
def _deg_kernel(src_hbm, dst_hbm, ones_hbm, z1_hbm,
                do_out, di_out, sidx0, didx0, sidx1, didx1, ones_v, z1_v,
                do_acc, di_acc, isem0, isem1):
    cid = lax.axis_index("c")
    sid = lax.axis_index("s")
    wid = cid * NS + sid

    pltpu.sync_copy(z1_hbm, z1_v)
    pltpu.sync_copy(z1_v, do_acc.at[pl.ds(sid * ZR, ZR)])
    pltpu.sync_copy(z1_v, di_acc.at[pl.ds(sid * ZR, ZR)])
    pltpu.sync_copy(ones_hbm, ones_v)
    plsc.subcore_barrier()

    start = (wid * TCHD) // NW
    end = ((wid + 1) * TCHD) // NW
    ncw = end - start
    nfull = ncw // NSL
    sidx = (sidx0, sidx1)
    didx = (didx0, didx1)
    isem = (isem0, isem1)

    def body(t, carry):
        c0 = start + t * NSL
        descs = []
        for k in range(NSL):
            off = pl.multiple_of((c0 + k) * KD, 8)
            descs.append(pltpu.async_copy(
                src_hbm.at[pl.ds(off, KD)], sidx[k], isem[k]))
            descs.append(pltpu.async_copy(
                dst_hbm.at[pl.ds(off, KD)], didx[k], isem[k]))
        for k in range(NSL):
            descs[2 * k].wait()
            descs[2 * k + 1].wait()
            pltpu.sync_copy(ones_v, do_acc.at[sidx[k]], add=True)
            pltpu.sync_copy(ones_v, di_acc.at[didx[k]], add=True)
        return carry

    lax.fori_loop(0, nfull, body, 0)

    @pl.when(ncw - nfull * NSL > 0)
    def _():
        off = pl.multiple_of((start + nfull * NSL) * KD, 8)
        pltpu.sync_copy(src_hbm.at[pl.ds(off, KD)], sidx0)
        pltpu.sync_copy(ones_v, do_acc.at[sidx0], add=True)
        pltpu.sync_copy(dst_hbm.at[pl.ds(off, KD)], didx0)
        pltpu.sync_copy(ones_v, di_acc.at[didx0], add=True)

    plsc.subcore_barrier()

    r0 = sid * ZR
    pltpu.sync_copy(do_acc.at[pl.ds(r0, ZR)], do_out.at[cid, pl.ds(r0, ZR)])
    pltpu.sync_copy(di_acc.at[pl.ds(r0, ZR)], di_out.at[cid, pl.ds(r0, ZR)])


@functools.partial(
    pl.kernel,
    out_type=[
        jax.ShapeDtypeStruct((NC, ACC, D), jnp.float32),
        jax.ShapeDtypeStruct((NC, ACC), jnp.float32),
    ],
    mesh=_mesh,
    scratch_types=[
        pltpu.VMEM((K,), jnp.int32),
        pltpu.VMEM((K,), jnp.int32),
        pltpu.VMEM((K,), jnp.int32),
        pltpu.VMEM((K,), jnp.int32),
        pltpu.VMEM((K, D), jnp.float32),
        pltpu.VMEM((K, D), jnp.float32),
        pltpu.VMEM((K,), jnp.float32),
        pltpu.VMEM((K,), jnp.float32),
        pltpu.VMEM((ZCH, D), jnp.float32),
        pltpu.VMEM((ZR,), jnp.float32),
        pltpu.VMEM_SHARED((ACC, D), jnp.float32),
        pltpu.VMEM_SHARED((ACC,), jnp.float32),
        pltpu.SemaphoreType.DMA,
        pltpu.SemaphoreType.DMA,
        pltpu.SemaphoreType.DMA,
        pltpu.SemaphoreType.DMA,
        pltpu.SemaphoreType.DMA,
        pltpu.SemaphoreType.DMA,
    ],
)
def _spmm_kernel(xp_hbm, ndst_hbm, src_hbm, dst_hbm, zrows_hbm, z1_hbm,
                 agg_out, c_out,
                 sidx0, didx0, sidx1, didx1, rows0, rows1, nv0, nv1,
                 zb_v, z1_v, agg_acc, c_acc,
                 isem0, isem1, gsem0, gsem1, ssem0, ssem1):
    cid = lax.axis_index("c")
    sid = lax.axis_index("s")
    wid = cid * NS + sid

    pltpu.sync_copy(zrows_hbm, zb_v)
    for z in range(ZR // ZCH):
        pltpu.sync_copy(zb_v, agg_acc.at[pl.ds(sid * ZR + z * ZCH, ZCH)])
    pltpu.sync_copy(z1_hbm, z1_v)
    pltpu.sync_copy(z1_v, c_acc.at[pl.ds(sid * ZR, ZR)])
    plsc.subcore_barrier()

    start = wid * CPW
    nfull = CPW // NSLS
    sidx = (sidx0, sidx1)
    didx = (didx0, didx1)
    rows = (rows0, rows1)
    nv = (nv0, nv1)
    isem = (isem0, isem1)
    gsem = (gsem0, gsem1)
    ssem = (ssem0, ssem1)

    def drain_scatter(k):
        pltpu.make_async_copy(rows[k], agg_acc.at[didx[k]], ssem[k]).wait()
        pltpu.make_async_copy(nv[k], c_acc.at[sidx[k]], ssem[k]).wait()

    def body(t, carry):
        @pl.when(t > 0)
        def _():
            for k in range(NSLS):
                drain_scatter(k)
        c0 = start + t * NSLS
        idescs = []
        for k in range(NSLS):
            off = pl.multiple_of((c0 + k) * K, 8)
            idescs.append(pltpu.async_copy(
                src_hbm.at[pl.ds(off, K)], sidx[k], isem[k]))
            idescs.append(pltpu.async_copy(
                dst_hbm.at[pl.ds(off, K)], didx[k], isem[k]))
        gdescs = []
        for k in range(NSLS):
            idescs[2 * k].wait()
            idescs[2 * k + 1].wait()
            gdescs.append(pltpu.async_copy(
                xp_hbm.at[sidx[k]], rows[k], gsem[k]))
            gdescs.append(pltpu.async_copy(
                ndst_hbm.at[didx[k]], nv[k], gsem[k]))
        for k in range(NSLS):
            gdescs[2 * k].wait()
            gdescs[2 * k + 1].wait()
            pltpu.async_copy(rows[k], agg_acc.at[didx[k]], ssem[k], add=True)
            pltpu.async_copy(nv[k], c_acc.at[sidx[k]], ssem[k], add=True)
        return carry

    lax.fori_loop(0, nfull, body, 0)
    for k in range(NSLS):
        drain_scatter(k)

    for j in range(CPW - (CPW // NSLS) * NSLS):
        off = pl.multiple_of((start + (CPW // NSLS) * NSLS + j) * K, 8)
        pltpu.sync_copy(src_hbm.at[pl.ds(off, K)], sidx0)
        pltpu.sync_copy(dst_hbm.at[pl.ds(off, K)], didx0)
        pltpu.async_copy(xp_hbm.at[sidx0], rows0, gsem0).wait()
        pltpu.sync_copy(ndst_hbm.at[didx0], nv0)
        pltpu.sync_copy(rows0, agg_acc.at[didx0], add=True)
        pltpu.sync_copy(nv0, c_acc.at[sidx0], add=True)

    plsc.subcore_barrier()

    for z in range(ZR // ZCH):
        r0 = sid * ZR + z * ZCH
        pltpu.sync_copy(agg_acc.at[pl.ds(r0, ZCH)],
                        agg_out.at[cid, pl.ds(r0, ZCH)])
    r0 = sid * ZR
    pltpu.sync_copy(c_acc.at[pl.ds(r0, ZR)], c_out.at[cid, pl.ds(r0, ZR)])


def _prep_body(do_ref, di_ref, x_ref, ns_ref, nd_ref, xp_ref):
    do = do_ref[0] + do_ref[1]
    di = di_ref[0] + di_ref[1]
    ns = lax.rsqrt(jnp.maximum(do, 1.0))
    ns_ref[...] = ns
    nd_ref[...] = lax.rsqrt(jnp.maximum(di, 1.0))
    xp_ref[...] = x_ref[...] * ns


BLK = 1280


def _finish_body(agg_ref, nd_ref, ns_ref, c_ref,
                 w1_ref, b1_ref, w2_ref, b2_ref, wc_ref, bc_ref,
                 out_ref, acc_ref):
    i = pl.program_id(0)

    @pl.when(i == 0)
    def _():
        acc_ref[...] = jnp.zeros_like(acc_ref)

    agg = (agg_ref[0] + agg_ref[1]) * nd_ref[...]
    h1 = jnp.dot(agg, w1_ref[...], preferred_element_type=jnp.float32)
    h1 = h1 + b1_ref[...]
    g = jnp.maximum(h1, 0.0) * ns_ref[...]
    cvec = c_ref[0] + c_ref[1]
    acc_ref[...] += jnp.sum(g * cvec, axis=0, keepdims=True)

    @pl.when(i == pl.num_programs(0) - 1)
    def _():
        pooled = acc_ref[...] * (1.0 / N)
        hg = jnp.dot(pooled, w2_ref[...], preferred_element_type=jnp.float32)
        hg = hg + b2_ref[...]
        out_ref[...] = (
            jnp.dot(hg, wc_ref[...], preferred_element_type=jnp.float32)
            + bc_ref[...])


def kernel(in_feat, edge_index, W1, b1, W2, b2, Wc, bc):
    src = edge_index[0]
    dst = edge_index[1]
    ones_k = jnp.ones((KD,), jnp.float32)
    z1 = jnp.zeros((ZR,), jnp.float32)
    zrows = jnp.zeros((ZCH, D), jnp.float32)

    deg_out_p, deg_in_p = _deg_kernel(src, dst, ones_k, z1)

    ns, nd, xp = pl.pallas_call(
        _prep_body,
        out_shape=[
            jax.ShapeDtypeStruct((ACC, 1), jnp.float32),
            jax.ShapeDtypeStruct((ACC, 1), jnp.float32),
            jax.ShapeDtypeStruct((ACC, D), jnp.float32),
        ],
    )(deg_out_p.reshape(NC, ACC, 1), deg_in_p.reshape(NC, ACC, 1),
      jnp.pad(in_feat, ((0, ACC - N), (0, 0))))

    agg_p, c_p = _spmm_kernel(xp, nd.reshape(ACC), src, dst, zrows, z1)

    wc_pad = jnp.pad(Wc, ((0, 0), (0, D - C)))
    bc_pad = jnp.pad(bc, ((0, D - C)))

    grid = ACC // BLK
    out_pad = pl.pallas_call(
        _finish_body,
        grid=(grid,),
        in_specs=[
            pl.BlockSpec((NC, BLK, D), lambda i: (0, i, 0)),
            pl.BlockSpec((BLK, 1), lambda i: (i, 0)),
            pl.BlockSpec((BLK, 1), lambda i: (i, 0)),
            pl.BlockSpec((NC, BLK, 1), lambda i: (0, i, 0)),
            pl.BlockSpec((D, H), lambda i: (0, 0)),
            pl.BlockSpec((1, H), lambda i: (0, 0)),
            pl.BlockSpec((H, H), lambda i: (0, 0)),
            pl.BlockSpec((1, H), lambda i: (0, 0)),
            pl.BlockSpec((H, D), lambda i: (0, 0)),
            pl.BlockSpec((1, D), lambda i: (0, 0)),
        ],
        out_specs=pl.BlockSpec((1, D), lambda i: (0, 0)),
        out_shape=jax.ShapeDtypeStruct((1, D), jnp.float32),
        scratch_shapes=[pltpu.VMEM((1, D), jnp.float32)],
    )(agg_p, nd, ns, c_p.reshape(NC, ACC, 1),
      W1, b1.reshape(1, H), W2, b2.reshape(1, H), wc_pad, bc_pad.reshape(1, D))

    return out_pad[:, :C]
